# baseline (speedup 1.0000x reference)
import functools

import jax
import jax.numpy as jnp
from jax import lax
from jax.experimental import pallas as pl
from jax.experimental.pallas import tpu as pltpu
from jax.experimental.pallas import tpu_sc as plsc

N_NODES = 10000
N_EDGES = 160000
D = 128
B = 2048

NC = 2
NS = 16
L = 16

N_PAD = 10240
E_PAD = 163840
EPT = E_PAD // NS
CHUNK = 128
NCH = EPT // CHUNK
ROWS_P = 512
NPH = B // (ROWS_P * NC)
SEG = ROWS_P * B // NS
ZCH = 1024
TRAIL = 4


def _sc_build(src_hbm, dst_hbm, lmap_hbm, bpgi_hbm, x_hbm,
              a_hbm, xsub_hbm,
              lmap_v, src_v, dst_v, idx_v, val_v,
              zero_v, bidx_v, rows_v, a_sp, sem, sem2, sem3):
  c = lax.axis_index("c")
  s = lax.axis_index("s")
  wid = c * NS + s

  rpw = B // (NC * NS)
  pltpu.sync_copy(bpgi_hbm.at[pl.ds(wid * rpw, rpw)], bidx_v)
  pltpu.async_copy(x_hbm.at[bidx_v], rows_v, sem).wait()
  pltpu.sync_copy(rows_v, xsub_hbm.at[pl.ds(wid * rpw, rpw)])

  pltpu.sync_copy(lmap_hbm, lmap_v)
  base = s * EPT
  pltpu.sync_copy(src_hbm.at[pl.ds(base, EPT)], src_v)
  pltpu.sync_copy(dst_hbm.at[pl.ds(base, EPT)], dst_v)

  def _zb(i, _):
    zero_v[pl.ds(i * L, L)] = jnp.zeros((L,), jnp.float32)
    return 0
  lax.fori_loop(0, ZCH // L, _zb, 0)

  def _tb(i, _):
    for k in range(CHUNK // L):
      sl_ = pl.ds(k * L, L)
      idx_v[NCH + i, sl_] = jnp.zeros((L,), jnp.int32)
      val_v[NCH + i, sl_] = jnp.zeros((L,), jnp.float32)
    return 0
  lax.fori_loop(0, TRAIL, _tb, 0)

  flush_cp = None
  for p in range(NPH):
    blk = p * NC + c
    lo = blk * (ROWS_P * B)

    if flush_cp is not None:
      flush_cp.wait()

    zero_cp = pltpu.async_copy(zero_v, a_sp.at[pl.ds(s * SEG, ZCH)], sem)
    def _zs(i, _):
      pltpu.async_copy(zero_v, a_sp.at[pl.ds(s * SEG + i * ZCH, ZCH)], sem)
      return 0
    lax.fori_loop(1, SEG // ZCH, _zs, 0)

    def _wb(j, _):
      for k in range(CHUNK // L):
        sl_ = pl.ds(k * L, L)
        off = pl.ds(j * CHUNK + k * L, L)
        sloc = plsc.load_gather(lmap_v, [src_v[off]])
        dloc = plsc.load_gather(lmap_v, [dst_v[off]])
        valid = (sloc >= 0) & (dloc >= 0)
        flat = jnp.where(valid, dloc, 0) * B + jnp.where(valid, sloc, 0)
        inr = valid & (flat >= lo) & (flat < lo + ROWS_P * B)
        idx_v[j, sl_] = jnp.where(inr, flat - lo, 0)
        val_v[j, sl_] = jnp.where(inr, 1.0, 0.0).astype(jnp.float32)
      return 0
    lax.fori_loop(0, NCH, _wb, 0)

    def _zw(i, _):
      zero_cp.wait()
      return 0
    lax.fori_loop(0, SEG // ZCH, _zw, 0)
    plsc.subcore_barrier()

    def _sb(g, _):
      cps = [
          pltpu.async_copy(val_v.at[g * 8 + t], a_sp.at[idx_v.at[g * 8 + t]],
                           sem2, add=True)
          for t in range(8)
      ]
      for cp in cps:
        cp.wait()
      return 0
    lax.fori_loop(0, NCH // 8, _sb, 0)
    tcps = [
        pltpu.async_copy(val_v.at[NCH + t], a_sp.at[idx_v.at[NCH + t]],
                         sem2, add=True)
        for t in range(TRAIL)
    ]
    for cp in tcps:
      cp.wait()
    plsc.subcore_barrier()

    flush_cp = pltpu.async_copy(a_sp.at[pl.ds(s * SEG, SEG)],
                                a_hbm.at[pl.ds(lo + s * SEG, SEG)], sem3)
  flush_cp.wait()


def _tc1_body(a_ref, xs_ref, w1_ref, b1_ref, h1_ref, adj_ref):
  a = a_ref[...]
  deg = jnp.maximum(jnp.sum(a, axis=1, keepdims=True), 1.0)
  ax = jnp.dot(a, xs_ref[...], preferred_element_type=jnp.float32)
  pre = jnp.dot(ax, w1_ref[...], preferred_element_type=jnp.float32)
  h1_ref[...] = jnp.maximum(pre / deg + b1_ref[...], 0.0)
  adj_ref[...] = jnp.minimum(a, 1.0)


def _tc2_body(a_ref, h1_ref, w2_ref, b2_ref, wmu_ref, bmu_ref, wlv_ref,
              blv_ref, wat_ref, bat_ref, wp1_ref, bp1_ref, wp2_ref, bp2_ref,
              mu_ref, lv_ref, rx_ref, mp_ref):
  a = a_ref[...]
  deg = jnp.maximum(jnp.sum(a, axis=1, keepdims=True), 1.0)
  ah = jnp.dot(a, h1_ref[...], preferred_element_type=jnp.float32)
  pre = jnp.dot(ah, w2_ref[...], preferred_element_type=jnp.float32)
  h2 = jnp.maximum(pre / deg + b2_ref[...], 0.0)
  mu = jnp.dot(h2, wmu_ref[...], preferred_element_type=jnp.float32) + bmu_ref[...]
  mu_ref[...] = mu
  lv_ref[...] = jnp.dot(h2, wlv_ref[...], preferred_element_type=jnp.float32) + blv_ref[...]
  rx_ref[...] = jnp.dot(mu, wat_ref[...], preferred_element_type=jnp.float32) + bat_ref[...]
  p1 = jnp.maximum(
      jnp.dot(mu, wp1_ref[...], preferred_element_type=jnp.float32) + bp1_ref[...], 0.0)
  mp_ref[...] = jnp.dot(p1, wp2_ref[...], preferred_element_type=jnp.float32) + bp2_ref[...]


def _tc3_body(mu_blk_ref, mu_all_ref, out_ref):
  out_ref[...] = lax.dot_general(
      mu_blk_ref[...], mu_all_ref[...], (((1,), (1,)), ((), ())),
      preferred_element_type=jnp.float32)


BLK = 256
GRID = B // BLK


def _full(shape):
  return pl.BlockSpec(shape, lambda i: (0,) * len(shape))


def kernel(x, edge_index, batch_patient_global_indices,
           W_gnn1, b_gnn1, W_gnn2, b_gnn2,
           W_mu, b_mu, W_lv, b_lv,
           W_attr, b_attr, W_p1, b_p1, W_p2, b_p2):
  src = edge_index[0].astype(jnp.int32)
  dst = edge_index[1].astype(jnp.int32)
  bpgi = batch_patient_global_indices.astype(jnp.int32)

  lmap = jnp.full((N_PAD,), -1, jnp.int32)
  lmap = lmap.at[bpgi].set(jnp.arange(B, dtype=jnp.int32))

  src_p = jnp.full((E_PAD,), N_NODES, jnp.int32).at[:N_EDGES].set(src)
  dst_p = jnp.full((E_PAD,), N_NODES, jnp.int32).at[:N_EDGES].set(dst)

  mesh = plsc.VectorSubcoreMesh(core_axis_name="c", subcore_axis_name="s")
  sc_fn = functools.partial(
      pl.kernel,
      out_type=(jax.ShapeDtypeStruct((B * B,), jnp.float32),
                jax.ShapeDtypeStruct((B, D), jnp.float32)),
      mesh=mesh,
      scratch_types=[
          pltpu.VMEM((N_PAD,), jnp.int32),
          pltpu.VMEM((EPT,), jnp.int32),
          pltpu.VMEM((EPT,), jnp.int32),
          pltpu.VMEM((NCH + TRAIL, CHUNK), jnp.int32),
          pltpu.VMEM((NCH + TRAIL, CHUNK), jnp.float32),
          pltpu.VMEM((ZCH,), jnp.float32),
          pltpu.VMEM((B // (NC * NS),), jnp.int32),
          pltpu.VMEM((B // (NC * NS), D), jnp.float32),
          pltpu.VMEM_SHARED((ROWS_P * B,), jnp.float32),
          pltpu.SemaphoreType.DMA,
          pltpu.SemaphoreType.DMA,
          pltpu.SemaphoreType.DMA,
      ],
      compiler_params=pltpu.CompilerParams(needs_layout_passes=False),
      name="gvae_sc_build",
  )(_sc_build)
  a_flat, x_sub = sc_fn(src_p, dst_p, lmap, bpgi, x)
  a = a_flat.reshape(B, B)

  b1 = b_gnn1.reshape(1, D)
  b2 = b_gnn2.reshape(1, D)
  bmu = b_mu.reshape(1, D)
  blv = b_lv.reshape(1, D)
  bat = b_attr.reshape(1, D)
  bp1 = b_p1.reshape(1, D)
  bp2 = b_p2.reshape(1, D)

  row_blk = pl.BlockSpec((BLK, B), lambda i: (i, 0))
  out_blk = pl.BlockSpec((BLK, D), lambda i: (i, 0))

  h1, orig_adj = pl.pallas_call(
      _tc1_body,
      grid=(GRID,),
      in_specs=[row_blk, _full((B, D)), _full((D, D)), _full((1, D))],
      out_specs=[out_blk, row_blk],
      out_shape=[jax.ShapeDtypeStruct((B, D), jnp.float32),
                 jax.ShapeDtypeStruct((B, B), jnp.float32)],
      name="gvae_tc_layer1",
  )(a, x_sub, W_gnn1, b1)

  mu, logvar, rec_x, mu_proj = pl.pallas_call(
      _tc2_body,
      grid=(GRID,),
      in_specs=[row_blk, _full((B, D))] + [_full((D, D)), _full((1, D))] * 6,
      out_specs=[out_blk] * 4,
      out_shape=[jax.ShapeDtypeStruct((B, D), jnp.float32)] * 4,
      name="gvae_tc_layer2_heads",
  )(a, h1, W_gnn2, b2, W_mu, bmu, W_lv, blv, W_attr, bat,
    W_p1, bp1, W_p2, bp2)

  rec_adj_logits = pl.pallas_call(
      _tc3_body,
      grid=(GRID,),
      in_specs=[out_blk, _full((B, D))],
      out_specs=row_blk,
      out_shape=jax.ShapeDtypeStruct((B, B), jnp.float32),
      name="gvae_tc_gram",
  )(mu, mu)

  return mu, logvar, rec_adj_logits, rec_x, mu_proj, orig_adj


# speedup vs baseline: 3.9854x; 3.9854x over previous
import functools

import jax
import jax.numpy as jnp
from jax import lax
from jax.experimental import pallas as pl
from jax.experimental.pallas import tpu as pltpu
from jax.experimental.pallas import tpu_sc as plsc

N_NODES = 10000
N_EDGES = 160000
D = 128
B = 2048

NC = 2
NS = 16
L = 16

N_PAD = 10240
E_PAD = 163840
EPT = E_PAD // NS
NVCH = EPT // L
ROWS_P = 512
NPH = B // (ROWS_P * NC)
SEG = ROWS_P * B // NS
ZCH = 2048
TRAIL = 8
JUNK_G = B * B
JUNK_L = ROWS_P * B


def _sc_build(src_hbm, dst_hbm, lmap_hbm, bpgi_hbm, x_hbm,
              a_hbm, xsub_hbm,
              lmap_v, src_v, dst_v, fidx_v, cidx_v,
              ones_v, zero_v, bidx_v, rows_v, a_sp, sem, sem2, sem3):
  c = lax.axis_index("c")
  s = lax.axis_index("s")
  wid = c * NS + s

  rpw = B // (NC * NS)
  pltpu.sync_copy(bpgi_hbm.at[pl.ds(wid * rpw, rpw)], bidx_v)
  pltpu.async_copy(x_hbm.at[bidx_v], rows_v, sem).wait()
  pltpu.sync_copy(rows_v, xsub_hbm.at[pl.ds(wid * rpw, rpw)])

  pltpu.sync_copy(lmap_hbm, lmap_v)
  base = s * EPT
  pltpu.sync_copy(src_hbm.at[pl.ds(base, EPT)], src_v)
  pltpu.sync_copy(dst_hbm.at[pl.ds(base, EPT)], dst_v)

  ones_v[...] = jnp.ones((L,), jnp.float32)
  def _zb(i, _):
    zero_v[pl.ds(i * L, L)] = jnp.zeros((L,), jnp.float32)
    return 0
  lax.fori_loop(0, ZCH // L, _zb, 0)

  def _cb(j, nv):
    off = pl.ds(j * L, L)
    sloc = plsc.load_gather(lmap_v, [src_v[off]])
    dloc = plsc.load_gather(lmap_v, [dst_v[off]])
    valid = (sloc >= 0) & (dloc >= 0)
    flat = dloc * B + sloc
    plsc.store_compressed(fidx_v.at[pl.ds(nv, L)], flat, mask=valid)
    return nv + jnp.sum(valid.astype(jnp.int32))
  nv = lax.fori_loop(0, NVCH, _cb, jnp.int32(0))
  fidx_v[pl.ds(nv, L)] = jnp.full((L,), JUNK_G, jnp.int32)
  nvch = (nv + (L - 1)) // L

  flush_cp = None
  for p in range(NPH):
    blk = p * NC + c
    lo = blk * (ROWS_P * B)

    if flush_cp is not None:
      flush_cp.wait()

    zero_cp = pltpu.async_copy(zero_v, a_sp.at[pl.ds(s * SEG, ZCH)], sem)
    def _zs(i, _):
      pltpu.async_copy(zero_v, a_sp.at[pl.ds(s * SEG + i * ZCH, ZCH)], sem)
      return 0
    lax.fori_loop(1, SEG // ZCH, _zs, 0)

    def _wb(i, co):
      f = fidx_v[pl.ds(i * L, L)]
      inr = (f >= lo) & (f < lo + ROWS_P * B)
      plsc.store_compressed(cidx_v.at[pl.ds(co, L)], f - lo, mask=inr)
      return co + jnp.sum(inr.astype(jnp.int32))
    co = lax.fori_loop(0, nvch, _wb, jnp.int32(0))
    junk = jnp.full((L,), JUNK_L, jnp.int32)
    for t in range(TRAIL + 1):
      cidx_v[pl.ds(co + t * L, L)] = junk

    def _zw(i, _):
      zero_cp.wait()
      return 0
    lax.fori_loop(0, SEG // ZCH, _zw, 0)
    plsc.subcore_barrier()

    ns = (co + (L - 1)) // L + TRAIL
    def _sb(g, _):
      idx = cidx_v[pl.ds(g * L, L)]
      pltpu.sync_copy(ones_v, a_sp.at[idx], add=True)
      return 0
    lax.fori_loop(0, ns, _sb, 0)
    plsc.subcore_barrier()

    flush_cp = pltpu.async_copy(a_sp.at[pl.ds(s * SEG, SEG)],
                                a_hbm.at[pl.ds(lo + s * SEG, SEG)], sem3)
  flush_cp.wait()


def _tc1_body(a_ref, xs_ref, w1_ref, b1_ref, h1_ref, adj_ref):
  a = a_ref[...]
  deg = jnp.maximum(jnp.sum(a, axis=1, keepdims=True), 1.0)
  ax = jnp.dot(a, xs_ref[...], preferred_element_type=jnp.float32)
  pre = jnp.dot(ax, w1_ref[...], preferred_element_type=jnp.float32)
  h1_ref[...] = jnp.maximum(pre / deg + b1_ref[...], 0.0)
  adj_ref[...] = jnp.minimum(a, 1.0)


def _tc2_body(a_ref, h1_ref, w2_ref, b2_ref, wmu_ref, bmu_ref, wlv_ref,
              blv_ref, wat_ref, bat_ref, wp1_ref, bp1_ref, wp2_ref, bp2_ref,
              mu_ref, lv_ref, rx_ref, mp_ref):
  a = a_ref[...]
  deg = jnp.maximum(jnp.sum(a, axis=1, keepdims=True), 1.0)
  ah = jnp.dot(a, h1_ref[...], preferred_element_type=jnp.float32)
  pre = jnp.dot(ah, w2_ref[...], preferred_element_type=jnp.float32)
  h2 = jnp.maximum(pre / deg + b2_ref[...], 0.0)
  mu = jnp.dot(h2, wmu_ref[...], preferred_element_type=jnp.float32) + bmu_ref[...]
  mu_ref[...] = mu
  lv_ref[...] = jnp.dot(h2, wlv_ref[...], preferred_element_type=jnp.float32) + blv_ref[...]
  rx_ref[...] = jnp.dot(mu, wat_ref[...], preferred_element_type=jnp.float32) + bat_ref[...]
  p1 = jnp.maximum(
      jnp.dot(mu, wp1_ref[...], preferred_element_type=jnp.float32) + bp1_ref[...], 0.0)
  mp_ref[...] = jnp.dot(p1, wp2_ref[...], preferred_element_type=jnp.float32) + bp2_ref[...]


def _tc3_body(mu_blk_ref, mu_all_ref, out_ref):
  out_ref[...] = lax.dot_general(
      mu_blk_ref[...], mu_all_ref[...], (((1,), (1,)), ((), ())),
      preferred_element_type=jnp.float32)


BLK = 256
GRID = B // BLK


def _full(shape):
  return pl.BlockSpec(shape, lambda i: (0,) * len(shape))


def kernel(x, edge_index, batch_patient_global_indices,
           W_gnn1, b_gnn1, W_gnn2, b_gnn2,
           W_mu, b_mu, W_lv, b_lv,
           W_attr, b_attr, W_p1, b_p1, W_p2, b_p2):
  src = edge_index[0].astype(jnp.int32)
  dst = edge_index[1].astype(jnp.int32)
  bpgi = batch_patient_global_indices.astype(jnp.int32)

  lmap = jnp.full((N_PAD,), -1, jnp.int32)
  lmap = lmap.at[bpgi].set(jnp.arange(B, dtype=jnp.int32))

  src_p = jnp.full((E_PAD,), N_NODES, jnp.int32).at[:N_EDGES].set(src)
  dst_p = jnp.full((E_PAD,), N_NODES, jnp.int32).at[:N_EDGES].set(dst)

  mesh = plsc.VectorSubcoreMesh(core_axis_name="c", subcore_axis_name="s")
  sc_fn = functools.partial(
      pl.kernel,
      out_type=(jax.ShapeDtypeStruct((B * B,), jnp.float32),
                jax.ShapeDtypeStruct((B, D), jnp.float32)),
      mesh=mesh,
      scratch_types=[
          pltpu.VMEM((N_PAD,), jnp.int32),
          pltpu.VMEM((EPT,), jnp.int32),
          pltpu.VMEM((EPT,), jnp.int32),
          pltpu.VMEM((EPT + L,), jnp.int32),
          pltpu.VMEM((EPT + (TRAIL + 2) * L,), jnp.int32),
          pltpu.VMEM((L,), jnp.float32),
          pltpu.VMEM((ZCH,), jnp.float32),
          pltpu.VMEM((B // (NC * NS),), jnp.int32),
          pltpu.VMEM((B // (NC * NS), D), jnp.float32),
          pltpu.VMEM_SHARED((ROWS_P * B + 8,), jnp.float32),
          pltpu.SemaphoreType.DMA,
          pltpu.SemaphoreType.DMA,
          pltpu.SemaphoreType.DMA,
      ],
      compiler_params=pltpu.CompilerParams(needs_layout_passes=False),
      name="gvae_sc_build",
  )(_sc_build)
  a_flat, x_sub = sc_fn(src_p, dst_p, lmap, bpgi, x)
  a = a_flat.reshape(B, B)

  b1 = b_gnn1.reshape(1, D)
  b2 = b_gnn2.reshape(1, D)
  bmu = b_mu.reshape(1, D)
  blv = b_lv.reshape(1, D)
  bat = b_attr.reshape(1, D)
  bp1 = b_p1.reshape(1, D)
  bp2 = b_p2.reshape(1, D)

  row_blk = pl.BlockSpec((BLK, B), lambda i: (i, 0))
  out_blk = pl.BlockSpec((BLK, D), lambda i: (i, 0))

  h1, orig_adj = pl.pallas_call(
      _tc1_body,
      grid=(GRID,),
      in_specs=[row_blk, _full((B, D)), _full((D, D)), _full((1, D))],
      out_specs=[out_blk, row_blk],
      out_shape=[jax.ShapeDtypeStruct((B, D), jnp.float32),
                 jax.ShapeDtypeStruct((B, B), jnp.float32)],
      name="gvae_tc_layer1",
  )(a, x_sub, W_gnn1, b1)

  mu, logvar, rec_x, mu_proj = pl.pallas_call(
      _tc2_body,
      grid=(GRID,),
      in_specs=[row_blk, _full((B, D))] + [_full((D, D)), _full((1, D))] * 6,
      out_specs=[out_blk] * 4,
      out_shape=[jax.ShapeDtypeStruct((B, D), jnp.float32)] * 4,
      name="gvae_tc_layer2_heads",
  )(a, h1, W_gnn2, b2, W_mu, bmu, W_lv, blv, W_attr, bat,
    W_p1, bp1, W_p2, bp2)

  rec_adj_logits = pl.pallas_call(
      _tc3_body,
      grid=(GRID,),
      in_specs=[out_blk, _full((B, D))],
      out_specs=row_blk,
      out_shape=jax.ShapeDtypeStruct((B, B), jnp.float32),
      name="gvae_tc_gram",
  )(mu, mu)

  return mu, logvar, rec_adj_logits, rec_x, mu_proj, orig_adj


# speedup vs baseline: 4.1452x; 1.0401x over previous
import functools

import jax
import jax.numpy as jnp
from jax import lax
from jax.experimental import pallas as pl
from jax.experimental.pallas import tpu as pltpu
from jax.experimental.pallas import tpu_sc as plsc

N_NODES = 10000
N_EDGES = 160000
D = 128
B = 2048

NC = 2
NS = 16
L = 16

EPT = N_EDGES // NS
NVCH = EPT // L
ROWS_P = 512
NPH = B // (ROWS_P * NC)
SEG = ROWS_P * B // NS
ZCH = 2048
TRAIL = 8
JUNK_G = B * B
JUNK_L = ROWS_P * B


def _sc_build(src_hbm, dst_hbm, lmap_hbm, bpgi_hbm, x_hbm,
              a_hbm, xsub_hbm,
              lmap_v, src_v, dst_v, fidx_v, cidx_v,
              ones_v, zero_v, bidx_v, rows_v, a_sp, sem, sem2, sem3):
  c = lax.axis_index("c")
  s = lax.axis_index("s")
  wid = c * NS + s

  rpw = B // (NC * NS)
  pltpu.sync_copy(bpgi_hbm.at[pl.ds(wid * rpw, rpw)], bidx_v)
  pltpu.async_copy(x_hbm.at[bidx_v], rows_v, sem).wait()
  pltpu.sync_copy(rows_v, xsub_hbm.at[pl.ds(wid * rpw, rpw)])

  pltpu.sync_copy(lmap_hbm, lmap_v)
  base = s * EPT
  pltpu.sync_copy(src_hbm.at[pl.ds(base, EPT)], src_v)
  pltpu.sync_copy(dst_hbm.at[pl.ds(base, EPT)], dst_v)

  ones_v[...] = jnp.ones((L,), jnp.float32)
  def _zb(i, _):
    zero_v[pl.ds(i * L, L)] = jnp.zeros((L,), jnp.float32)
    return 0
  lax.fori_loop(0, ZCH // L, _zb, 0)

  def _cb(j, nv):
    off = pl.ds(j * L, L)
    sloc = plsc.load_gather(lmap_v, [src_v[off]])
    dloc = plsc.load_gather(lmap_v, [dst_v[off]])
    valid = (sloc >= 0) & (dloc >= 0)
    flat = dloc * B + sloc
    plsc.store_compressed(fidx_v.at[pl.ds(nv, L)], flat, mask=valid)
    return nv + jnp.sum(valid.astype(jnp.int32))
  nv = lax.fori_loop(0, NVCH, _cb, jnp.int32(0))
  fidx_v[pl.ds(nv, L)] = jnp.full((L,), JUNK_G, jnp.int32)
  nvch = (nv + (L - 1)) // L

  flush_cp = None
  for p in range(NPH):
    blk = p * NC + c
    lo = blk * (ROWS_P * B)

    if flush_cp is not None:
      flush_cp.wait()

    zero_cp = pltpu.async_copy(zero_v, a_sp.at[pl.ds(s * SEG, ZCH)], sem)
    def _zs(i, _):
      pltpu.async_copy(zero_v, a_sp.at[pl.ds(s * SEG + i * ZCH, ZCH)], sem)
      return 0
    lax.fori_loop(1, SEG // ZCH, _zs, 0)

    def _wb(i, co):
      f = fidx_v[pl.ds(i * L, L)]
      inr = (f >= lo) & (f < lo + ROWS_P * B)
      plsc.store_compressed(cidx_v.at[pl.ds(co, L)], f - lo, mask=inr)
      return co + jnp.sum(inr.astype(jnp.int32))
    co = lax.fori_loop(0, nvch, _wb, jnp.int32(0))
    junk = jnp.full((L,), JUNK_L, jnp.int32)
    for t in range(TRAIL + 1):
      cidx_v[pl.ds(co + t * L, L)] = junk

    def _zw(i, _):
      zero_cp.wait()
      return 0
    lax.fori_loop(0, SEG // ZCH, _zw, 0)
    plsc.subcore_barrier()

    ns = (co + (L - 1)) // L + TRAIL
    def _sb(g, _):
      idx = cidx_v[pl.ds(g * L, L)]
      pltpu.sync_copy(ones_v, a_sp.at[idx], add=True)
      return 0
    lax.fori_loop(0, ns, _sb, 0)
    plsc.subcore_barrier()

    flush_cp = pltpu.async_copy(a_sp.at[pl.ds(s * SEG, SEG)],
                                a_hbm.at[pl.ds(lo + s * SEG, SEG)], sem3)
  flush_cp.wait()


BLK = 256
GRID = B // BLK


def _tc_body(a_ref, xs_ref, w1_ref, b1_ref, w2_ref, b2_ref, wmu_ref, bmu_ref,
             wlv_ref, blv_ref, wat_ref, bat_ref, wp1_ref, bp1_ref, wp2_ref,
             bp2_ref,
             adj_ref, mu_ref, lv_ref, rx_ref, mp_ref, gram_ref,
             h1_scr, mu_scr):
  i = pl.program_id(0)

  @pl.when(i < GRID)
  def _layer1():
    a = a_ref[...]
    deg = jnp.maximum(jnp.sum(a, axis=1, keepdims=True), 1.0)
    ax = jnp.dot(a, xs_ref[...], preferred_element_type=jnp.float32)
    pre = jnp.dot(ax, w1_ref[...], preferred_element_type=jnp.float32)
    h1_scr[pl.ds(i * BLK, BLK), :] = jnp.maximum(pre / deg + b1_ref[...], 0.0)
    adj_ref[...] = jnp.minimum(a, 1.0)

  @pl.when((i >= GRID) & (i < 2 * GRID))
  def _layer2_heads():
    a = a_ref[...]
    deg = jnp.maximum(jnp.sum(a, axis=1, keepdims=True), 1.0)
    ah = jnp.dot(a, h1_scr[...], preferred_element_type=jnp.float32)
    pre = jnp.dot(ah, w2_ref[...], preferred_element_type=jnp.float32)
    h2 = jnp.maximum(pre / deg + b2_ref[...], 0.0)
    mu = jnp.dot(h2, wmu_ref[...], preferred_element_type=jnp.float32) + bmu_ref[...]
    mu_scr[pl.ds((i - GRID) * BLK, BLK), :] = mu
    mu_ref[...] = mu
    lv_ref[...] = jnp.dot(h2, wlv_ref[...], preferred_element_type=jnp.float32) + blv_ref[...]
    rx_ref[...] = jnp.dot(mu, wat_ref[...], preferred_element_type=jnp.float32) + bat_ref[...]
    p1 = jnp.maximum(
        jnp.dot(mu, wp1_ref[...], preferred_element_type=jnp.float32) + bp1_ref[...], 0.0)
    mp_ref[...] = jnp.dot(p1, wp2_ref[...], preferred_element_type=jnp.float32) + bp2_ref[...]

  @pl.when(i >= 2 * GRID)
  def _gram():
    mu_blk = mu_scr[pl.ds((i - 2 * GRID) * BLK, BLK), :]
    gram_ref[...] = lax.dot_general(
        mu_blk, mu_scr[...], (((1,), (1,)), ((), ())),
        preferred_element_type=jnp.float32)


def _full(shape):
  return pl.BlockSpec(shape, lambda i: (0,) * len(shape))


def kernel(x, edge_index, batch_patient_global_indices,
           W_gnn1, b_gnn1, W_gnn2, b_gnn2,
           W_mu, b_mu, W_lv, b_lv,
           W_attr, b_attr, W_p1, b_p1, W_p2, b_p2):
  src = edge_index[0].astype(jnp.int32)
  dst = edge_index[1].astype(jnp.int32)
  bpgi = batch_patient_global_indices.astype(jnp.int32)

  lmap = jnp.full((N_NODES,), -1, jnp.int32)
  lmap = lmap.at[bpgi].set(jnp.arange(B, dtype=jnp.int32))

  mesh = plsc.VectorSubcoreMesh(core_axis_name="c", subcore_axis_name="s")
  sc_fn = functools.partial(
      pl.kernel,
      out_type=(jax.ShapeDtypeStruct((B * B,), jnp.float32),
                jax.ShapeDtypeStruct((B, D), jnp.float32)),
      mesh=mesh,
      scratch_types=[
          pltpu.VMEM((N_NODES,), jnp.int32),
          pltpu.VMEM((EPT,), jnp.int32),
          pltpu.VMEM((EPT,), jnp.int32),
          pltpu.VMEM((EPT + L,), jnp.int32),
          pltpu.VMEM((EPT + (TRAIL + 2) * L,), jnp.int32),
          pltpu.VMEM((L,), jnp.float32),
          pltpu.VMEM((ZCH,), jnp.float32),
          pltpu.VMEM((B // (NC * NS),), jnp.int32),
          pltpu.VMEM((B // (NC * NS), D), jnp.float32),
          pltpu.VMEM_SHARED((ROWS_P * B + 8,), jnp.float32),
          pltpu.SemaphoreType.DMA,
          pltpu.SemaphoreType.DMA,
          pltpu.SemaphoreType.DMA,
      ],
      compiler_params=pltpu.CompilerParams(needs_layout_passes=False),
      name="gvae_sc_build",
  )(_sc_build)
  a_flat, x_sub = sc_fn(src, dst, lmap, bpgi, x)
  a = a_flat.reshape(B, B)

  b1 = b_gnn1.reshape(1, D)
  b2 = b_gnn2.reshape(1, D)
  bmu = b_mu.reshape(1, D)
  blv = b_lv.reshape(1, D)
  bat = b_attr.reshape(1, D)
  bp1 = b_p1.reshape(1, D)
  bp2 = b_p2.reshape(1, D)

  a_spec = pl.BlockSpec((BLK, B),
                        lambda i: (jnp.where(i < 2 * GRID, i % GRID, GRID - 1), 0))
  adj_spec = pl.BlockSpec((BLK, B), lambda i: (jnp.minimum(i, GRID - 1), 0))
  head_spec = pl.BlockSpec((BLK, D),
                           lambda i: (jnp.clip(i - GRID, 0, GRID - 1), 0))
  gram_spec = pl.BlockSpec((BLK, B),
                           lambda i: (jnp.clip(i - 2 * GRID, 0, GRID - 1), 0))

  orig_adj, mu, logvar, rec_x, mu_proj, rec_adj_logits = pl.pallas_call(
      _tc_body,
      grid=(3 * GRID,),
      in_specs=[a_spec, _full((B, D)), _full((D, D)), _full((1, D)),
                _full((D, D)), _full((1, D)), _full((D, D)), _full((1, D)),
                _full((D, D)), _full((1, D)), _full((D, D)), _full((1, D)),
                _full((D, D)), _full((1, D)), _full((D, D)), _full((1, D))],
      out_specs=[adj_spec, head_spec, head_spec, head_spec, head_spec,
                 gram_spec],
      out_shape=[jax.ShapeDtypeStruct((B, B), jnp.float32)] +
                [jax.ShapeDtypeStruct((B, D), jnp.float32)] * 4 +
                [jax.ShapeDtypeStruct((B, B), jnp.float32)],
      scratch_shapes=[pltpu.VMEM((B, D), jnp.float32),
                      pltpu.VMEM((B, D), jnp.float32)],
      name="gvae_tc_dense",
  )(a, x_sub, W_gnn1, b1, W_gnn2, b2, W_mu, bmu, W_lv, blv,
    W_attr, bat, W_p1, bp1, W_p2, bp2)

  return mu, logvar, rec_adj_logits, rec_x, mu_proj, orig_adj


# speedup vs baseline: 4.3669x; 1.0535x over previous
import functools

import jax
import jax.numpy as jnp
from jax import lax
from jax.experimental import pallas as pl
from jax.experimental.pallas import tpu as pltpu
from jax.experimental.pallas import tpu_sc as plsc

N_NODES = 10000
N_EDGES = 160000
D = 128
B = 2048

NC = 2
NS = 16
L = 16

EPT = N_EDGES // NS
NVCH = EPT // L
ROWS_P = 512
NPH = B // (ROWS_P * NC)
SEG = ROWS_P * B // NS
ZCH = 2048
TRAIL = 8
JUNK_G = B * B
JUNK_L = ROWS_P * B


def _sc_build(src_hbm, dst_hbm, lmap_hbm, bpgi_hbm, x_hbm,
              a_hbm, xsub_hbm,
              lmap_v, src_v, dst_v, fidx_v, cidx_v,
              ones_v, zero_v, bidx_v, rows_v, a_sp, sem, sem2, sem3):
  c = lax.axis_index("c")
  s = lax.axis_index("s")
  wid = c * NS + s

  rpw = B // (NC * NS)
  pltpu.sync_copy(bpgi_hbm.at[pl.ds(wid * rpw, rpw)], bidx_v)
  pltpu.async_copy(x_hbm.at[bidx_v], rows_v, sem).wait()
  pltpu.sync_copy(rows_v, xsub_hbm.at[pl.ds(wid * rpw, rpw)])

  pltpu.sync_copy(lmap_hbm, lmap_v)
  base = s * EPT
  pltpu.sync_copy(src_hbm.at[pl.ds(base, EPT)], src_v)
  pltpu.sync_copy(dst_hbm.at[pl.ds(base, EPT)], dst_v)

  ones_v[...] = jnp.ones((L,), jnp.float32)
  def _zb(i, _):
    zero_v[pl.ds(i * L, L)] = jnp.zeros((L,), jnp.float32)
    return 0
  lax.fori_loop(0, ZCH // L, _zb, 0)

  def _cb(j, nv):
    for u in range(2):
      off = pl.ds((j * 2 + u) * L, L)
      sloc = plsc.load_gather(lmap_v, [src_v[off]])
      dloc = plsc.load_gather(lmap_v, [dst_v[off]])
      valid = (sloc >= 0) & (dloc >= 0)
      flat = dloc * B + sloc
      plsc.store_compressed(fidx_v.at[pl.ds(nv, L)], flat, mask=valid)
      nv = nv + jnp.sum(valid.astype(jnp.int32))
    return nv
  nv = lax.fori_loop(0, NVCH // 2, _cb, jnp.int32(0))
  if NVCH % 2:
    off = pl.ds((NVCH - 1) * L, L)
    sloc = plsc.load_gather(lmap_v, [src_v[off]])
    dloc = plsc.load_gather(lmap_v, [dst_v[off]])
    valid = (sloc >= 0) & (dloc >= 0)
    plsc.store_compressed(fidx_v.at[pl.ds(nv, L)], dloc * B + sloc, mask=valid)
    nv = nv + jnp.sum(valid.astype(jnp.int32))
  fidx_v[pl.ds(nv, L)] = jnp.full((L,), JUNK_G, jnp.int32)
  nvch = (nv + (L - 1)) // L

  flush_cp = None
  for p in range(NPH):
    blk = p * NC + c
    lo = blk * (ROWS_P * B)

    if flush_cp is not None:
      flush_cp.wait()

    zero_cp = pltpu.async_copy(zero_v, a_sp.at[pl.ds(s * SEG, ZCH)], sem)
    def _zs(i, _):
      pltpu.async_copy(zero_v, a_sp.at[pl.ds(s * SEG + i * ZCH, ZCH)], sem)
      return 0
    lax.fori_loop(1, SEG // ZCH, _zs, 0)

    def _wb(i, co):
      f = fidx_v[pl.ds(i * L, L)]
      inr = (f >= lo) & (f < lo + ROWS_P * B)
      plsc.store_compressed(cidx_v.at[pl.ds(co, L)], f - lo, mask=inr)
      return co + jnp.sum(inr.astype(jnp.int32))
    co = lax.fori_loop(0, nvch, _wb, jnp.int32(0))
    junk = jnp.full((L,), JUNK_L, jnp.int32)
    for t in range(TRAIL + 1):
      cidx_v[pl.ds(co + t * L, L)] = junk

    def _zw(i, _):
      zero_cp.wait()
      return 0
    lax.fori_loop(0, SEG // ZCH, _zw, 0)
    plsc.subcore_barrier()

    ns = (co + (L - 1)) // L + TRAIL
    def _sb(g, _):
      idx = cidx_v[pl.ds(g * L, L)]
      pltpu.sync_copy(ones_v, a_sp.at[idx], add=True)
      return 0
    lax.fori_loop(0, ns, _sb, 0)
    plsc.subcore_barrier()

    flush_cp = pltpu.async_copy(a_sp.at[pl.ds(s * SEG, SEG)],
                                a_hbm.at[pl.ds(lo + s * SEG, SEG)], sem3)
  flush_cp.wait()


BLK = 512
GRID = B // BLK


def _tc_body(a_ref, xs_ref, w1_ref, b1_ref, w2_ref, b2_ref, wmu_ref, bmu_ref,
             wlv_ref, blv_ref, wat_ref, bat_ref, wp1_ref, bp1_ref, wp2_ref,
             bp2_ref,
             adj_ref, mu_ref, lv_ref, rx_ref, mp_ref, gram_ref,
             h1_scr, mu_scr):
  i = pl.program_id(0)

  @pl.when(i < GRID)
  def _layer1():
    a = a_ref[...]
    deg = jnp.maximum(jnp.sum(a, axis=1, keepdims=True), 1.0)
    ax = jnp.dot(a, xs_ref[...], preferred_element_type=jnp.float32)
    pre = jnp.dot(ax, w1_ref[...], preferred_element_type=jnp.float32)
    h1_scr[pl.ds(i * BLK, BLK), :] = jnp.maximum(pre / deg + b1_ref[...], 0.0)
    adj_ref[...] = jnp.minimum(a, 1.0)

  @pl.when((i >= GRID) & (i < 2 * GRID))
  def _layer2_heads():
    a = a_ref[...]
    deg = jnp.maximum(jnp.sum(a, axis=1, keepdims=True), 1.0)
    ah = jnp.dot(a, h1_scr[...], preferred_element_type=jnp.float32)
    pre = jnp.dot(ah, w2_ref[...], preferred_element_type=jnp.float32)
    h2 = jnp.maximum(pre / deg + b2_ref[...], 0.0)
    mu = jnp.dot(h2, wmu_ref[...], preferred_element_type=jnp.float32) + bmu_ref[...]
    mu_scr[pl.ds((i - GRID) * BLK, BLK), :] = mu
    mu_ref[...] = mu
    lv_ref[...] = jnp.dot(h2, wlv_ref[...], preferred_element_type=jnp.float32) + blv_ref[...]
    rx_ref[...] = jnp.dot(mu, wat_ref[...], preferred_element_type=jnp.float32) + bat_ref[...]
    p1 = jnp.maximum(
        jnp.dot(mu, wp1_ref[...], preferred_element_type=jnp.float32) + bp1_ref[...], 0.0)
    mp_ref[...] = jnp.dot(p1, wp2_ref[...], preferred_element_type=jnp.float32) + bp2_ref[...]

  @pl.when(i >= 2 * GRID)
  def _gram():
    mu_blk = mu_scr[pl.ds((i - 2 * GRID) * BLK, BLK), :]
    gram_ref[...] = lax.dot_general(
        mu_blk, mu_scr[...], (((1,), (1,)), ((), ())),
        preferred_element_type=jnp.float32)


def _full(shape):
  return pl.BlockSpec(shape, lambda i: (0,) * len(shape))


def kernel(x, edge_index, batch_patient_global_indices,
           W_gnn1, b_gnn1, W_gnn2, b_gnn2,
           W_mu, b_mu, W_lv, b_lv,
           W_attr, b_attr, W_p1, b_p1, W_p2, b_p2):
  src = edge_index[0].astype(jnp.int32)
  dst = edge_index[1].astype(jnp.int32)
  bpgi = batch_patient_global_indices.astype(jnp.int32)

  lmap = jnp.full((N_NODES,), -1, jnp.int32)
  lmap = lmap.at[bpgi].set(jnp.arange(B, dtype=jnp.int32))

  mesh = plsc.VectorSubcoreMesh(core_axis_name="c", subcore_axis_name="s")
  sc_fn = functools.partial(
      pl.kernel,
      out_type=(jax.ShapeDtypeStruct((B * B,), jnp.float32),
                jax.ShapeDtypeStruct((B, D), jnp.float32)),
      mesh=mesh,
      scratch_types=[
          pltpu.VMEM((N_NODES,), jnp.int32),
          pltpu.VMEM((EPT,), jnp.int32),
          pltpu.VMEM((EPT,), jnp.int32),
          pltpu.VMEM((EPT + L,), jnp.int32),
          pltpu.VMEM((EPT + (TRAIL + 2) * L,), jnp.int32),
          pltpu.VMEM((L,), jnp.float32),
          pltpu.VMEM((ZCH,), jnp.float32),
          pltpu.VMEM((B // (NC * NS),), jnp.int32),
          pltpu.VMEM((B // (NC * NS), D), jnp.float32),
          pltpu.VMEM_SHARED((ROWS_P * B + 8,), jnp.float32),
          pltpu.SemaphoreType.DMA,
          pltpu.SemaphoreType.DMA,
          pltpu.SemaphoreType.DMA,
      ],
      compiler_params=pltpu.CompilerParams(needs_layout_passes=False),
      name="gvae_sc_build",
  )(_sc_build)
  a_flat, x_sub = sc_fn(src, dst, lmap, bpgi, x)
  a = a_flat.reshape(B, B)

  b1 = b_gnn1.reshape(1, D)
  b2 = b_gnn2.reshape(1, D)
  bmu = b_mu.reshape(1, D)
  blv = b_lv.reshape(1, D)
  bat = b_attr.reshape(1, D)
  bp1 = b_p1.reshape(1, D)
  bp2 = b_p2.reshape(1, D)

  a_spec = pl.BlockSpec((BLK, B),
                        lambda i: (jnp.where(i < 2 * GRID, i % GRID, GRID - 1), 0))
  adj_spec = pl.BlockSpec((BLK, B), lambda i: (jnp.minimum(i, GRID - 1), 0))
  head_spec = pl.BlockSpec((BLK, D),
                           lambda i: (jnp.clip(i - GRID, 0, GRID - 1), 0))
  gram_spec = pl.BlockSpec((BLK, B),
                           lambda i: (jnp.clip(i - 2 * GRID, 0, GRID - 1), 0))

  orig_adj, mu, logvar, rec_x, mu_proj, rec_adj_logits = pl.pallas_call(
      _tc_body,
      grid=(3 * GRID,),
      in_specs=[a_spec, _full((B, D)), _full((D, D)), _full((1, D)),
                _full((D, D)), _full((1, D)), _full((D, D)), _full((1, D)),
                _full((D, D)), _full((1, D)), _full((D, D)), _full((1, D)),
                _full((D, D)), _full((1, D)), _full((D, D)), _full((1, D))],
      out_specs=[adj_spec, head_spec, head_spec, head_spec, head_spec,
                 gram_spec],
      out_shape=[jax.ShapeDtypeStruct((B, B), jnp.float32)] +
                [jax.ShapeDtypeStruct((B, D), jnp.float32)] * 4 +
                [jax.ShapeDtypeStruct((B, B), jnp.float32)],
      scratch_shapes=[pltpu.VMEM((B, D), jnp.float32),
                      pltpu.VMEM((B, D), jnp.float32)],
      name="gvae_tc_dense",
  )(a, x_sub, W_gnn1, b1, W_gnn2, b2, W_mu, bmu, W_lv, blv,
    W_attr, bat, W_p1, bp1, W_p2, bp2)

  return mu, logvar, rec_adj_logits, rec_x, mu_proj, orig_adj
